# baseline (speedup 1.0000x reference)
import jax
import jax.numpy as jnp
from jax import lax
from jax.experimental import pallas as pl
from jax.experimental.pallas import tpu as pltpu
from jax.experimental.pallas import tpu_sc as plsc

B = 16384
D = 32
RPB = 4
NC = 2
NS = 16
L = 16
NW = NC * NS
BPW = B // NW
NCH = 4
CH = BPW // NCH
GCH = CH // L


def _body(uidx_hbm, iidx_hbm, table_hbm, out_hbm,
          uidx_v, iidx_v, ublk_v, iblk_v, urows_v, irows_v, out_v, sems):
    wid = lax.axis_index("s") * NC + lax.axis_index("c")
    base = wid * BPW

    pltpu.sync_copy(uidx_hbm.at[pl.ds(base, BPW)], uidx_v)
    pltpu.sync_copy(iidx_hbm.at[pl.ds(base, BPW)], iidx_v)

    @plsc.parallel_loop(0, BPW // L)
    def blk(k):
        sl = pl.ds(k * L, L)
        ublk_v[sl] = lax.shift_right_logical(uidx_v[sl], 2)
        iblk_v[sl] = lax.shift_right_logical(iidx_v[sl], 2)

    def start(c):
        buf = c % 2
        cu = pltpu.async_copy(
            table_hbm.at[ublk_v.at[pl.ds(c * CH, CH)]],
            urows_v.at[buf], sems.at[buf, 0])
        ci = pltpu.async_copy(
            table_hbm.at[iblk_v.at[pl.ds(c * CH, CH)]],
            irows_v.at[buf], sems.at[buf, 1])
        return cu, ci

    lane = lax.iota(jnp.int32, L)
    three = jnp.full((L,), RPB - 1, jnp.int32)
    inflight = start(0)

    for c in range(NCH):
        if c + 1 < NCH:
            nxt = start(c + 1)
        cu, ci = inflight
        cu.wait()
        ci.wait()
        buf = c % 2

        @plsc.parallel_loop(0, GCH)
        def group(g):
            rid = g * L + lane
            gsl = pl.ds(c * CH + g * L, L)
            uoff = lax.shift_left(uidx_v[gsl] & three, 5)
            ioff = lax.shift_left(iidx_v[gsl] & three, 5)
            acc = jnp.zeros((L,), jnp.float32)
            for d in range(D):
                uv = plsc.load_gather(urows_v.at[buf], [rid, uoff + d])
                iv = plsc.load_gather(irows_v.at[buf], [rid, ioff + d])
                acc = acc + uv * iv
            out_v[gsl] = acc

        if c + 1 < NCH:
            inflight = nxt

    pltpu.sync_copy(out_v, out_hbm.at[pl.ds(base, BPW)])


def kernel(edge_index, edge_label_index, embedding_weight):
    del edge_index
    uidx = edge_label_index[0]
    iidx = edge_label_index[1]
    table = embedding_weight.reshape(-1, RPB * D)
    mesh = plsc.VectorSubcoreMesh(core_axis_name="c", subcore_axis_name="s")
    f = pl.kernel(
        _body,
        out_type=jax.ShapeDtypeStruct((B,), jnp.float32),
        mesh=mesh,
        compiler_params=pltpu.CompilerParams(needs_layout_passes=False),
        scratch_types=[
            pltpu.VMEM((BPW,), jnp.int32),
            pltpu.VMEM((BPW,), jnp.int32),
            pltpu.VMEM((BPW,), jnp.int32),
            pltpu.VMEM((BPW,), jnp.int32),
            pltpu.VMEM((2, CH, RPB * D), jnp.float32),
            pltpu.VMEM((2, CH, RPB * D), jnp.float32),
            pltpu.VMEM((BPW,), jnp.float32),
            pltpu.SemaphoreType.DMA((2, 2)),
        ],
    )
    return f(uidx, iidx, table)
